# baseline (speedup 1.0000x reference)
import jax
import jax.numpy as jnp
from jax.experimental import pallas as pl
from jax.experimental.pallas import tpu as pltpu


def _gelu(x):
    return 0.5 * x * (1.0 + jnp.tanh(0.7978845608028654 * (x + 0.044715 * x * x * x)))


def _suffix_sum(row, s):
    padded = jnp.concatenate([row, jnp.zeros_like(row)], axis=1)
    d = 1
    while d < s:
        padded = padded + pltpu.roll(padded, 2 * s - d, 1)
        d *= 2
    return padded[:, :s]


def _stats_kernel(x_ref, mr_ref, W0_ref, b0_ref, w1_ref,
                  W2_ref, b2_ref, w3_ref, cols_ref, arow_ref):
    x = x_ref[0]
    s = x.shape[0]
    w1 = w1_ref[...]
    w3 = w3_ref[...]
    mrowf = mr_ref[0].astype(jnp.float32)
    mrowb = mrowf > 0.0

    xw = jnp.dot(x.astype(jnp.bfloat16), W0_ref[...].astype(jnp.bfloat16),
                 preferred_element_type=jnp.float32)
    h = _gelu(xw + b0_ref[...])

    cdims = (((1,), (1,)), ((), ()))
    sl = jax.lax.dot_general(w1, h, cdims, preferred_element_type=jnp.float32)
    a_row = jax.lax.dot_general(w3, h, cdims,
                                preferred_element_type=jnp.float32)
    sc = _gelu(jnp.dot(h, W2_ref[...],
                       preferred_element_type=jnp.float32) + b2_ref[...])
    c_row = jax.lax.dot_general(w3, sc, cdims,
                                preferred_element_type=jnp.float32)

    slm = mrowf * sl + (mrowf - 1.0) * 10.0
    m1 = jnp.max(slm)
    z1 = jnp.sum(jnp.exp(slm - m1))
    slp = (m1 + jnp.log(z1)) - slm

    neg = jnp.float32(-1e30)
    ma = jnp.max(jnp.where(mrowb, a_row, neg))
    mc = jnp.max(jnp.where(mrowb, c_row, neg))
    m2 = jnp.maximum(ma + mc, -10.0)
    ea = jnp.where(mrowb, jnp.exp(a_row - ma), 0.0)
    ec = jnp.where(mrowb, jnp.exp(c_row - mc), 0.0)
    sa = _suffix_sum(ea, s)
    z2p = jnp.sum(ec * sa)
    p = jnp.sum(mrowf)
    npairs = 0.5 * p * (p + 1.0)
    z2 = z2p * jnp.exp((ma + mc) - m2) \
        + (s * s - npairs) * jnp.exp(-10.0 - m2)
    lse2 = m2 + jnp.log(z2)

    ut_row = slp + (lse2 + 10.0)
    d_row = c_row + 10.0

    pad = jnp.zeros_like(mrowf)
    stack = jnp.concatenate(
        [ut_row, d_row, mrowf, pad, pad, pad, pad, pad], axis=0)
    cols_ref[0] = jnp.transpose(stack, (1, 0))
    arow_ref[0] = a_row


def _emit_kernel(cols_ref, arow_ref, mr_ref, out_ref):
    q = pl.program_id(1)
    ch = out_ref.shape[1]
    s = out_ref.shape[2]
    cols = cols_ref[0]
    ut_c = cols[:, 0:1]
    d_c = cols[:, 1:2]
    mcolb = cols[:, 2:3] > 0.0
    a_row = arow_ref[0]
    mrowb = mr_ref[0].astype(jnp.float32) > 0.0

    ii = jax.lax.broadcasted_iota(jnp.int32, (ch, s), 0) + q * ch
    jj = jax.lax.broadcasted_iota(jnp.int32, (ch, s), 1)
    vb = (jj >= ii) & (mcolb & mrowb)
    out_ref[0] = ut_c - jnp.where(vb, d_c + a_row, 0.0)


@jax.jit
def kernel(inputs, mask, W0, b0, w1, W2, b2, w3):
    B, S, D = inputs.shape
    U = W0.shape[1]
    CH = 128
    mr = mask.reshape(B, 1, S)
    cols, arow = pl.pallas_call(
        _stats_kernel,
        grid=(B,),
        in_specs=[
            pl.BlockSpec((1, S, D), lambda b: (b, 0, 0)),
            pl.BlockSpec((1, 1, S), lambda b: (b, 0, 0)),
            pl.BlockSpec((D, U), lambda b: (0, 0)),
            pl.BlockSpec((1, U), lambda b: (0, 0)),
            pl.BlockSpec((1, U), lambda b: (0, 0)),
            pl.BlockSpec((U, U), lambda b: (0, 0)),
            pl.BlockSpec((1, U), lambda b: (0, 0)),
            pl.BlockSpec((1, U), lambda b: (0, 0)),
        ],
        out_specs=[
            pl.BlockSpec((1, S, 8), lambda b: (b, 0, 0)),
            pl.BlockSpec((1, 1, S), lambda b: (b, 0, 0)),
        ],
        out_shape=[
            jax.ShapeDtypeStruct((B, S, 8), jnp.float32),
            jax.ShapeDtypeStruct((B, 1, S), jnp.float32),
        ],
    )(inputs, mr, W0, b0.reshape(1, U), w1.reshape(1, U),
      W2, b2.reshape(1, U), w3.reshape(1, U))

    return pl.pallas_call(
        _emit_kernel,
        grid=(B, S // CH),
        in_specs=[
            pl.BlockSpec((1, CH, 8), lambda b, q: (b, q, 0)),
            pl.BlockSpec((1, 1, S), lambda b, q: (b, 0, 0)),
            pl.BlockSpec((1, 1, S), lambda b, q: (b, 0, 0)),
        ],
        out_specs=pl.BlockSpec((1, CH, S), lambda b, q: (b, q, 0)),
        out_shape=jax.ShapeDtypeStruct((B, S, S), jnp.float32),
    )(cols, arow, mr)


# speedup vs baseline: 1.5704x; 1.5704x over previous
import jax
import jax.numpy as jnp
from jax.experimental import pallas as pl


def _gelu(x):
    return 0.5 * x * (1.0 + jnp.tanh(0.7978845608028654 * (x + 0.044715 * x * x * x)))


def _answer_finder_kernel(x_ref, mr_ref, W0_ref, b0_ref, w1_ref,
                          W2_ref, b2_ref, w3_ref, out_ref):
    x = x_ref[0]
    s = out_ref.shape[1]
    w1 = w1_ref[...]
    w3 = w3_ref[...]
    mrowf = mr_ref[0].astype(jnp.float32)
    mrowb = mrowf > 0.0

    xw = jnp.dot(x.astype(jnp.bfloat16), W0_ref[...].astype(jnp.bfloat16),
                 preferred_element_type=jnp.float32)
    h = _gelu(xw + b0_ref[...])

    cdims = (((1,), (1,)), ((), ()))
    sl = jax.lax.dot_general(w1, h, cdims, preferred_element_type=jnp.float32)
    a_row = jax.lax.dot_general(w3, h, cdims,
                                preferred_element_type=jnp.float32)
    sc = _gelu(jnp.dot(h, W2_ref[...],
                       preferred_element_type=jnp.float32) + b2_ref[...])
    c_row = jax.lax.dot_general(w3, sc, cdims,
                                preferred_element_type=jnp.float32)

    slm = mrowf * sl + (mrowf - 1.0) * 10.0
    m1 = jnp.max(slm)
    z1 = jnp.sum(jnp.exp(slm - m1))
    slp = (m1 + jnp.log(z1)) - slm

    neg = jnp.float32(-1e30)
    ma = jnp.max(jnp.where(mrowb, a_row, neg))
    mc = jnp.max(jnp.where(mrowb, c_row, neg))
    m2 = jnp.maximum(ma + mc, -10.0)
    ea = jnp.where(mrowb, jnp.exp(a_row - ma), 0.0)
    ec = jnp.where(mrowb, jnp.exp(c_row - mc), 0.0)

    pad = jnp.zeros_like(mrowf)
    stack = jnp.concatenate(
        [ea, mrowf, ec, pad, pad, pad, pad, pad], axis=0)
    colsT = jnp.transpose(stack, (1, 0))
    ea_c = colsT[:, 0:1]
    mcolb = colsT[:, 1:2] > 0.0
    ec_c = colsT[:, 2:3]

    ii = jax.lax.broadcasted_iota(jnp.int32, (s, s), 0)
    jj = jax.lax.broadcasted_iota(jnp.int32, (s, s), 1)
    trib = jj >= ii
    tri_f = jnp.where(trib, 1.0, 0.0)
    sa_c = jax.lax.dot_general(tri_f, ea_c, (((1,), (0,)), ((), ())),
                               preferred_element_type=jnp.float32)
    z2p = jnp.sum(ec_c * sa_c)
    p = jnp.sum(mrowf)
    npairs = 0.5 * p * (p + 1.0)
    z2 = z2p * jnp.exp((ma + mc) - m2) \
        + (s * s - npairs) * jnp.exp(-10.0 - m2)
    lse2 = m2 + jnp.log(z2)

    ut_row = slp + (lse2 + 10.0)
    d_row = c_row + 10.0
    stack2 = jnp.concatenate(
        [ut_row, d_row, pad, pad, pad, pad, pad, pad], axis=0)
    cols2 = jnp.transpose(stack2, (1, 0))
    ut_c = cols2[:, 0:1]
    d_c = cols2[:, 1:2]

    vb = trib & (mcolb & mrowb)
    out_ref[0] = ut_c - jnp.where(vb, d_c + a_row, 0.0)


@jax.jit
def kernel(inputs, mask, W0, b0, w1, W2, b2, w3):
    B, S, D = inputs.shape
    U = W0.shape[1]
    mr = mask.reshape(B, 1, S)
    return pl.pallas_call(
        _answer_finder_kernel,
        grid=(B,),
        in_specs=[
            pl.BlockSpec((1, S, D), lambda b: (b, 0, 0)),
            pl.BlockSpec((1, 1, S), lambda b: (b, 0, 0)),
            pl.BlockSpec((D, U), lambda b: (0, 0)),
            pl.BlockSpec((1, U), lambda b: (0, 0)),
            pl.BlockSpec((1, U), lambda b: (0, 0)),
            pl.BlockSpec((U, U), lambda b: (0, 0)),
            pl.BlockSpec((1, U), lambda b: (0, 0)),
            pl.BlockSpec((1, U), lambda b: (0, 0)),
        ],
        out_specs=pl.BlockSpec((1, S, S), lambda b: (b, 0, 0)),
        out_shape=jax.ShapeDtypeStruct((B, S, S), jnp.float32),
    )(inputs, mr, W0, b0.reshape(1, U), w1.reshape(1, U),
      W2, b2.reshape(1, U), w3.reshape(1, U))


# speedup vs baseline: 1.6786x; 1.0689x over previous
import jax
import jax.numpy as jnp
from jax.experimental import pallas as pl


def _gelu(x):
    return 0.5 * x * (1.0 + jnp.tanh(0.7978845608028654 * (x + 0.044715 * x * x * x)))


def _one_batch(h, mrowf, w1, w3, W2, b2, tri_f, trib, iis, jjs):
    s = h.shape[0]
    mrowb = mrowf > 0.0

    cdims = (((1,), (1,)), ((), ()))
    sl = jax.lax.dot_general(w1, h, cdims, preferred_element_type=jnp.float32)
    a_row = jax.lax.dot_general(w3, h, cdims,
                                preferred_element_type=jnp.float32)
    sc = _gelu(jnp.dot(h, W2, preferred_element_type=jnp.float32) + b2)
    c_row = jax.lax.dot_general(w3, sc, cdims,
                                preferred_element_type=jnp.float32)

    slm = mrowf * sl + (mrowf - 1.0) * 10.0
    m1 = jnp.max(slm)
    z1 = jnp.sum(jnp.exp(slm - m1))
    slp = (m1 + jnp.log(z1)) - slm

    neg = jnp.float32(-1e30)
    ma = jnp.max(jnp.where(mrowb, a_row, neg))
    mc = jnp.max(jnp.where(mrowb, c_row, neg))
    m2 = jnp.maximum(ma + mc, -10.0)
    ea = jnp.where(mrowb, jnp.exp(a_row - ma), 0.0)
    ec = jnp.where(mrowb, jnp.exp(c_row - mc), 0.0)

    pad = jnp.zeros_like(mrowf)
    stack = jnp.concatenate(
        [ea, mrowf, ec, pad, pad, pad, pad, pad], axis=0)
    colsT = jnp.transpose(stack, (1, 0))
    ea_c = colsT[:, 0:1]
    mcolb = colsT[:, 1:2] > 0.0
    ec_c = colsT[:, 2:3]

    sa_c = jax.lax.dot_general(tri_f, ea_c, (((1,), (0,)), ((), ())),
                               preferred_element_type=jnp.float32)
    z2p = jnp.sum(ec_c * sa_c)
    p = jnp.sum(mrowf)
    npairs = 0.5 * p * (p + 1.0)
    z2 = z2p * jnp.exp((ma + mc) - m2) \
        + (s * s - npairs) * jnp.exp(-10.0 - m2)
    lse2 = m2 + jnp.log(z2)

    ut_row = slp + (lse2 + 10.0)
    d_row = c_row + 10.0
    stack2 = jnp.concatenate(
        [ut_row, d_row, pad, pad, pad, pad, pad, pad], axis=0)
    cols2 = jnp.transpose(stack2, (1, 0))
    ut_c = cols2[:, 0:1]
    d_c = cols2[:, 1:2]

    vb = trib & (mcolb & mrowb)
    return ut_c - jnp.where(vb, d_c + a_row, 0.0)


def _answer_finder_kernel(x_ref, mr_ref, W0_ref, b0_ref, w1_ref,
                          W2_ref, b2_ref, w3_ref, out_ref):
    nb = x_ref.shape[0]
    s = out_ref.shape[1]
    w1 = w1_ref[...]
    w3 = w3_ref[...]
    W2 = W2_ref[...]
    b2 = b2_ref[...]

    xall = x_ref[...].reshape(nb * s, x_ref.shape[2])
    hall = _gelu(jnp.dot(xall.astype(jnp.bfloat16),
                         W0_ref[...].astype(jnp.bfloat16),
                         preferred_element_type=jnp.float32) + b0_ref[...])

    ii = jax.lax.broadcasted_iota(jnp.int32, (s, s), 0)
    jj = jax.lax.broadcasted_iota(jnp.int32, (s, s), 1)
    trib = jj >= ii
    tri_f = jnp.where(trib, 1.0, 0.0)

    for bb in range(nb):
        h = hall[bb * s:(bb + 1) * s, :]
        mrowf = mr_ref[bb].astype(jnp.float32)
        out_ref[bb] = _one_batch(h, mrowf, w1, w3, W2, b2,
                                 tri_f, trib, ii, jj)


@jax.jit
def kernel(inputs, mask, W0, b0, w1, W2, b2, w3):
    B, S, D = inputs.shape
    U = W0.shape[1]
    NB = 2
    mr = mask.reshape(B, 1, S)
    return pl.pallas_call(
        _answer_finder_kernel,
        grid=(B // NB,),
        in_specs=[
            pl.BlockSpec((NB, S, D), lambda b: (b, 0, 0)),
            pl.BlockSpec((NB, 1, S), lambda b: (b, 0, 0)),
            pl.BlockSpec((D, U), lambda b: (0, 0)),
            pl.BlockSpec((1, U), lambda b: (0, 0)),
            pl.BlockSpec((1, U), lambda b: (0, 0)),
            pl.BlockSpec((U, U), lambda b: (0, 0)),
            pl.BlockSpec((1, U), lambda b: (0, 0)),
            pl.BlockSpec((1, U), lambda b: (0, 0)),
        ],
        out_specs=pl.BlockSpec((NB, S, S), lambda b: (b, 0, 0)),
        out_shape=jax.ShapeDtypeStruct((B, S, S), jnp.float32),
    )(inputs, mr, W0, b0.reshape(1, U), w1.reshape(1, U),
      W2, b2.reshape(1, U), w3.reshape(1, U))
